# baseline (speedup 1.0000x reference)
import jax
import jax.numpy as jnp
from jax.experimental import pallas as pl
from jax.experimental.pallas import tpu as pltpu

_N_COPIES = 16


def _copy_kernel(in_ref, out_ref, sems):
    rows = in_ref.shape[0]
    block = rows // _N_COPIES
    for i in range(_N_COPIES):
        pltpu.make_async_copy(
            in_ref.at[pl.ds(i * block, block)],
            out_ref.at[pl.ds(i * block, block)],
            sems.at[i],
        ).start()
    for i in range(_N_COPIES):
        pltpu.make_async_copy(
            in_ref.at[pl.ds(i * block, block)],
            out_ref.at[pl.ds(i * block, block)],
            sems.at[i],
        ).wait()


def kernel(x, embedding):
    seq_len = x.shape[1]
    features = embedding.shape[1]
    return pl.pallas_call(
        _copy_kernel,
        in_specs=[pl.BlockSpec(memory_space=pl.ANY)],
        out_specs=pl.BlockSpec(memory_space=pl.ANY),
        scratch_shapes=[pltpu.SemaphoreType.DMA((_N_COPIES,))],
        out_shape=jax.ShapeDtypeStruct((seq_len, features), embedding.dtype),
    )(embedding)


# speedup vs baseline: 44.6162x; 44.6162x over previous
import jax
import jax.numpy as jnp
from jax.experimental import pallas as pl

_BLOCK = 1024


def _copy_block(in_ref, out_ref):
    out_ref[...] = in_ref[...]


def kernel(x, embedding):
    seq_len = x.shape[1]
    features = embedding.shape[1]
    return pl.pallas_call(
        _copy_block,
        grid=(seq_len // _BLOCK,),
        in_specs=[pl.BlockSpec((_BLOCK, features), lambda i: (i, 0))],
        out_specs=pl.BlockSpec((_BLOCK, features), lambda i: (i, 0)),
        out_shape=jax.ShapeDtypeStruct((seq_len, features), embedding.dtype),
    )(embedding)


# speedup vs baseline: 48.1240x; 1.0786x over previous
import jax
import jax.numpy as jnp
from jax.experimental import pallas as pl

_BLOCK = 2048


def _copy_block(in_ref, out_ref):
    out_ref[...] = in_ref[...]


def kernel(x, embedding):
    seq_len = x.shape[1]
    features = embedding.shape[1]
    return pl.pallas_call(
        _copy_block,
        grid=(seq_len // _BLOCK,),
        in_specs=[pl.BlockSpec((_BLOCK, features), lambda i: (i, 0))],
        out_specs=pl.BlockSpec((_BLOCK, features), lambda i: (i, 0)),
        out_shape=jax.ShapeDtypeStruct((seq_len, features), embedding.dtype),
    )(embedding)
